# baseline (speedup 1.0000x reference)
import functools

import jax
import jax.numpy as jnp
from jax.experimental import pallas as pl
from jax.experimental.pallas import tpu as pltpu
from jax.experimental.pallas import tpu_sc as plsc

_BATCH = 64
_TOPK = 512
_DIM = 1024
_HID = 2048
_MEM = 128
_BB1 = 8
_BB2 = 4
_NSLICE = 4
_SB = _BATCH // _NSLICE
_NIDX_S = _SB * _MEM
_NC = 2
_NW = 32
_BPW = _NIDX_S // _NW


def _topk_kernel(w_ref, idx_ref, sw_ref):
    w = jnp.maximum(w_ref[...], 0.0)
    s = jnp.maximum(jnp.sum(w, axis=2, keepdims=True), 1e-6)
    wn = w / s

    wn_col = jnp.transpose(wn, (0, 2, 1))
    wi = jnp.broadcast_to(wn_col, (_BB1, _TOPK, _TOPK))
    wj = jnp.broadcast_to(wn, (_BB1, _TOPK, _TOPK))
    ii = jax.lax.broadcasted_iota(jnp.int32, (1, _TOPK, _TOPK), 1)
    jj = jax.lax.broadcasted_iota(jnp.int32, (1, _TOPK, _TOPK), 2)
    tie_ok = ii < jj
    beats = (wi > wj) | ((wi == wj) & tie_ok)
    rank = jnp.sum(beats.astype(jnp.int32), axis=1, keepdims=True)

    t_iota = jax.lax.broadcasted_iota(jnp.int32, (_BB1, _MEM, _TOPK), 1)
    eq = jnp.broadcast_to(rank, (_BB1, _MEM, _TOPK)) == t_iota

    sw_ref[...] = jnp.sum(
        jnp.where(eq, jnp.broadcast_to(wn, (_BB1, _MEM, _TOPK)), 0.0),
        axis=2)

    j_i = jax.lax.broadcasted_iota(jnp.int32, (_BB1, _MEM, _TOPK), 2)
    idx_local = jnp.sum(jnp.where(eq, j_i, 0), axis=2)
    base = (pl.program_id(0) * _BB1
            + jax.lax.broadcasted_iota(jnp.int32, (_BB1, _MEM), 0)) * _TOPK
    idx_ref[...] = idx_local + base


def _run_topk(w_slice):
    return pl.pallas_call(
        _topk_kernel,
        grid=(_SB // _BB1,),
        in_specs=[pl.BlockSpec((_BB1, 1, _TOPK), lambda i: (i, 0, 0))],
        out_specs=[
            pl.BlockSpec((_BB1, _MEM), lambda i: (i, 0)),
            pl.BlockSpec((_BB1, _MEM), lambda i: (i, 0)),
        ],
        out_shape=[
            jax.ShapeDtypeStruct((_SB, _MEM), jnp.int32),
            jax.ShapeDtypeStruct((_SB, _MEM), jnp.float32),
        ],
        compiler_params=pltpu.CompilerParams(
            dimension_semantics=("arbitrary",),
        ),
    )(w_slice)


def _sc_gather(e_flat, idx_flat):
    mesh = plsc.VectorSubcoreMesh(core_axis_name="c", subcore_axis_name="s")

    @functools.partial(
        pl.kernel, mesh=mesh,
        out_type=jax.ShapeDtypeStruct((_NIDX_S, _DIM), jnp.float32),
        scratch_types=[
            pltpu.VMEM((_BPW,), jnp.int32),
            pltpu.VMEM((_BPW, _DIM), jnp.float32),
            pltpu.SemaphoreType.DMA,
        ],
    )
    def k(table_hbm, idx_hbm, out_hbm, idx_v, rows_v, sem):
        wid = jax.lax.axis_index("s") * _NC + jax.lax.axis_index("c")
        base = wid * _BPW
        pltpu.sync_copy(idx_hbm.at[pl.ds(base, _BPW)], idx_v)
        pltpu.async_copy(table_hbm.at[idx_v], rows_v, sem).wait()
        pltpu.sync_copy(rows_v, out_hbm.at[pl.ds(base, _BPW)])

    return k(e_flat, idx_flat)


def _proj_kernel(sel_ref, sw_ref, wt_ref, b_ref, g_ref, bt_ref, out_ref):
    sel = jnp.clip(sel_ref[...], -5.0, 5.0)
    tokens = jax.lax.dot_general(
        sel, wt_ref[...], (((1,), (1,)), ((), ())),
        preferred_element_type=jnp.float32,
        precision=jax.lax.Precision.DEFAULT)
    tokens = (tokens + b_ref[...]) * sw_ref[...]
    tokens = jnp.clip(tokens, -5.0, 5.0)
    mean = jnp.mean(tokens, axis=-1, keepdims=True)
    cent = tokens - mean
    var = jnp.mean(cent * cent, axis=-1, keepdims=True)
    out = cent * jax.lax.rsqrt(var + 1e-5) * g_ref[...] + bt_ref[...]
    out_ref[...] = out.reshape(_BB2, _MEM, _HID)


def _run_proj(sel, sw2, W, b2, g2, bt2, s, out_prev):
    specs = [
        pl.BlockSpec((_BB2 * _MEM, _DIM), lambda i: (i, 0)),
        pl.BlockSpec((_BB2 * _MEM, 1), lambda i: (i, 0)),
        pl.BlockSpec((_HID, _DIM), lambda i: (0, 0)),
        pl.BlockSpec((1, _HID), lambda i: (0, 0)),
        pl.BlockSpec((1, _HID), lambda i: (0, 0)),
        pl.BlockSpec((1, _HID), lambda i: (0, 0)),
    ]
    args = [sel, sw2, W, b2, g2, bt2]
    kwargs = {}
    kern = _proj_kernel
    if out_prev is not None:
        specs.append(pl.BlockSpec(memory_space=pl.ANY))
        args.append(out_prev)
        kwargs["input_output_aliases"] = {6: 0}
        kern = lambda *refs: _proj_kernel(*refs[:6], refs[7])
    base = s * (_SB // _BB2)
    return pl.pallas_call(
        kern,
        grid=(_SB // _BB2,),
        in_specs=specs,
        out_specs=pl.BlockSpec((_BB2, _MEM, _HID),
                               lambda i: (i + base, 0, 0)),
        out_shape=jax.ShapeDtypeStruct((_BATCH, _MEM, _HID), jnp.float32),
        compiler_params=pltpu.CompilerParams(
            dimension_semantics=("arbitrary",),
        ),
        **kwargs,
    )(*args)


@jax.jit
def kernel(image_embeds, weights, W, b, gamma, beta):
    b2 = b.reshape(1, _HID)
    g2 = gamma.reshape(1, _HID)
    bt2 = beta.reshape(1, _HID)
    w3 = weights.reshape(_BATCH, 1, _TOPK)
    e_flat = image_embeds.reshape(_BATCH * _TOPK, _DIM)

    sels, sws = [], []
    for s in range(_NSLICE):
        idx, sw = _run_topk(
            jax.lax.slice_in_dim(w3, s * _SB, (s + 1) * _SB, axis=0))
        sels.append(_sc_gather(
            e_flat, (idx + s * _SB * _TOPK).reshape(_NIDX_S)))
        sws.append(sw.reshape(_NIDX_S, 1))

    out = None
    for s in range(_NSLICE):
        out = _run_proj(sels[s], sws[s], W, b2, g2, bt2, s, out)
    return out


# speedup vs baseline: 1.1630x; 1.1630x over previous
import functools

import jax
import jax.numpy as jnp
from jax.experimental import pallas as pl
from jax.experimental.pallas import tpu as pltpu
from jax.experimental.pallas import tpu_sc as plsc

_BATCH = 64
_TOPK = 512
_DIM = 1024
_HID = 2048
_MEM = 128
_BB1 = 8
_BB2 = 4
_NSLICE = 2
_SB = _BATCH // _NSLICE
_NIDX_S = _SB * _MEM
_NC = 2
_NW = 32
_BPW = _NIDX_S // _NW
_CH = 64
_NCH = _BPW // _CH


def _topk_kernel(w_ref, idx_ref, sw_ref):
    w = jnp.maximum(w_ref[...], 0.0)
    s = jnp.maximum(jnp.sum(w, axis=2, keepdims=True), 1e-6)
    wn = w / s

    wn_col = jnp.transpose(wn, (0, 2, 1))
    wi = jnp.broadcast_to(wn_col, (_BB1, _TOPK, _TOPK))
    wj = jnp.broadcast_to(wn, (_BB1, _TOPK, _TOPK))
    ii = jax.lax.broadcasted_iota(jnp.int32, (1, _TOPK, _TOPK), 1)
    jj = jax.lax.broadcasted_iota(jnp.int32, (1, _TOPK, _TOPK), 2)
    tie_ok = ii < jj
    beats = (wi > wj) | ((wi == wj) & tie_ok)
    rank = jnp.sum(beats.astype(jnp.int32), axis=1, keepdims=True)

    t_iota = jax.lax.broadcasted_iota(jnp.int32, (_BB1, _MEM, _TOPK), 1)
    eq = jnp.broadcast_to(rank, (_BB1, _MEM, _TOPK)) == t_iota

    sw_ref[...] = jnp.sum(
        jnp.where(eq, jnp.broadcast_to(wn, (_BB1, _MEM, _TOPK)), 0.0),
        axis=2)

    j_i = jax.lax.broadcasted_iota(jnp.int32, (_BB1, _MEM, _TOPK), 2)
    idx_local = jnp.sum(jnp.where(eq, j_i, 0), axis=2)
    base = (pl.program_id(0) * _BB1
            + jax.lax.broadcasted_iota(jnp.int32, (_BB1, _MEM), 0)) * _TOPK
    idx_ref[...] = idx_local + base


def _run_topk(w_slice):
    return pl.pallas_call(
        _topk_kernel,
        grid=(_SB // _BB1,),
        in_specs=[pl.BlockSpec((_BB1, 1, _TOPK), lambda i: (i, 0, 0))],
        out_specs=[
            pl.BlockSpec((_BB1, _MEM), lambda i: (i, 0)),
            pl.BlockSpec((_BB1, _MEM), lambda i: (i, 0)),
        ],
        out_shape=[
            jax.ShapeDtypeStruct((_SB, _MEM), jnp.int32),
            jax.ShapeDtypeStruct((_SB, _MEM), jnp.float32),
        ],
        compiler_params=pltpu.CompilerParams(
            dimension_semantics=("arbitrary",),
        ),
    )(w_slice)


def _sc_gather(e_flat, idx_flat):
    mesh = plsc.VectorSubcoreMesh(core_axis_name="c", subcore_axis_name="s")

    @functools.partial(
        pl.kernel, mesh=mesh,
        out_type=jax.ShapeDtypeStruct((_NIDX_S, _DIM), jnp.float32),
        scratch_types=[
            pltpu.VMEM((_CH,), jnp.int32),
            pltpu.VMEM((_CH, _DIM), jnp.float32),
            pltpu.SemaphoreType.DMA,
        ],
    )
    def k(table_hbm, idx_hbm, out_hbm, idx_v, rows_v, sem):
        wid = jax.lax.axis_index("s") * _NC + jax.lax.axis_index("c")
        base = wid * _BPW

        @pl.loop(0, _NCH)
        def _(c):
            off = base + c * _CH
            pltpu.sync_copy(idx_hbm.at[pl.ds(off, _CH)], idx_v)
            pltpu.async_copy(table_hbm.at[idx_v], rows_v, sem).wait()
            pltpu.sync_copy(rows_v, out_hbm.at[pl.ds(off, _CH)])

    return k(e_flat, idx_flat)


def _proj_kernel(sel_ref, sw_ref, wt_ref, b_ref, g_ref, bt_ref, out_ref):
    sel = jnp.clip(sel_ref[...], -5.0, 5.0)
    tokens = jax.lax.dot_general(
        sel, wt_ref[...], (((1,), (1,)), ((), ())),
        preferred_element_type=jnp.float32,
        precision=jax.lax.Precision.DEFAULT)
    tokens = (tokens + b_ref[...]) * sw_ref[...]
    tokens = jnp.clip(tokens, -5.0, 5.0)
    mean = jnp.mean(tokens, axis=-1, keepdims=True)
    cent = tokens - mean
    var = jnp.mean(cent * cent, axis=-1, keepdims=True)
    out = cent * jax.lax.rsqrt(var + 1e-5) * g_ref[...] + bt_ref[...]
    out_ref[...] = out.reshape(_BB2, _MEM, _HID)


def _run_proj(sel, sw2, W, b2, g2, bt2, s, out_prev):
    specs = [
        pl.BlockSpec((_BB2 * _MEM, _DIM), lambda i: (i, 0)),
        pl.BlockSpec((_BB2 * _MEM, 1), lambda i: (i, 0)),
        pl.BlockSpec((_HID, _DIM), lambda i: (0, 0)),
        pl.BlockSpec((1, _HID), lambda i: (0, 0)),
        pl.BlockSpec((1, _HID), lambda i: (0, 0)),
        pl.BlockSpec((1, _HID), lambda i: (0, 0)),
    ]
    args = [sel, sw2, W, b2, g2, bt2]
    kwargs = {}
    kern = _proj_kernel
    if out_prev is not None:
        specs.append(pl.BlockSpec(memory_space=pl.ANY))
        args.append(out_prev)
        kwargs["input_output_aliases"] = {6: 0}
        kern = lambda *refs: _proj_kernel(*refs[:6], refs[7])
    base = s * (_SB // _BB2)
    return pl.pallas_call(
        kern,
        grid=(_SB // _BB2,),
        in_specs=specs,
        out_specs=pl.BlockSpec((_BB2, _MEM, _HID),
                               lambda i: (i + base, 0, 0)),
        out_shape=jax.ShapeDtypeStruct((_BATCH, _MEM, _HID), jnp.float32),
        compiler_params=pltpu.CompilerParams(
            dimension_semantics=("arbitrary",),
        ),
        **kwargs,
    )(*args)


@jax.jit
def kernel(image_embeds, weights, W, b, gamma, beta):
    b2 = b.reshape(1, _HID)
    g2 = gamma.reshape(1, _HID)
    bt2 = beta.reshape(1, _HID)
    w3 = weights.reshape(_BATCH, 1, _TOPK)
    e_flat = image_embeds.reshape(_BATCH * _TOPK, _DIM)

    sels, sws = [], []
    for s in range(_NSLICE):
        idx, sw = _run_topk(
            jax.lax.slice_in_dim(w3, s * _SB, (s + 1) * _SB, axis=0))
        sels.append(_sc_gather(
            e_flat, (idx + s * _SB * _TOPK).reshape(_NIDX_S)))
        sws.append(sw.reshape(_NIDX_S, 1))

    out = None
    for s in range(_NSLICE):
        out = _run_proj(sels[s], sws[s], W, b2, g2, bt2, s, out)
    return out
